# baseline (speedup 1.0000x reference)
import functools

import jax
import jax.numpy as jnp
from jax import lax
from jax.experimental import pallas as pl
from jax.experimental.pallas import tpu as pltpu
from jax.experimental.pallas import tpu_sc as plsc

VOCAB = 1000000
EMBED = 64
B = 16384
L = 20
ROWS = B * L

TBLK = 2000
PAIRS = VOCAB // 2

NC = 2
NS = 16
NW = NC * NS
ROWS_PER_W = ROWS // NW
CHUNK = 128
K = 4
NITER = ROWS_PER_W // (K * CHUNK)


def _tf_body(xl_ref, xr_ref, w_ref, b2_ref, o_ref):
    w = w_ref[...]
    yl = lax.dot_general(xl_ref[...], w, (((1,), (1,)), ((), ())),
                         preferred_element_type=jnp.float32)
    yr = lax.dot_general(xr_ref[...], w, (((1,), (1,)), ((), ())),
                         preferred_element_type=jnp.float32)
    o_ref[...] = jnp.concatenate([yl, yr], axis=1) + b2_ref[...]


_transform = pl.pallas_call(
    _tf_body,
    grid=(VOCAB // (2 * TBLK),),
    in_specs=[
        pl.BlockSpec((TBLK, EMBED), lambda i: (2 * i, 0)),
        pl.BlockSpec((TBLK, EMBED), lambda i: (2 * i + 1, 0)),
        pl.BlockSpec((EMBED, EMBED), lambda i: (0, 0)),
        pl.BlockSpec((1, 128), lambda i: (0, 0)),
    ],
    out_specs=pl.BlockSpec((TBLK, 128), lambda i: (i, 0)),
    out_shape=jax.ShapeDtypeStruct((PAIRS, 128), jnp.float32),
)

_mesh = plsc.VectorSubcoreMesh(core_axis_name="c", subcore_axis_name="s")


@functools.partial(
    pl.kernel,
    mesh=_mesh,
    out_type=jax.ShapeDtypeStruct((ROWS, 128), jnp.float32),
    scratch_types=[
        pltpu.VMEM((K * CHUNK,), jnp.int32),
        pltpu.VMEM((K * CHUNK, 128), jnp.float32),
        pltpu.SemaphoreType.DMA,
    ],
    compiler_params=pltpu.CompilerParams(use_tc_tiling_on_sc=False),
)
def _sc_gather(table_hbm, idx_hbm, out_hbm, idx_v, rows_v, sem):
    wid = lax.axis_index("s") * NC + lax.axis_index("c")
    base = wid * ROWS_PER_W

    def body(i, carry):
        off = pl.multiple_of(base + i * (K * CHUNK), K * CHUNK)
        pltpu.sync_copy(idx_hbm.at[pl.ds(off, K * CHUNK)], idx_v)
        copies = [
            pltpu.async_copy(
                table_hbm.at[idx_v.at[pl.ds(k * CHUNK, CHUNK)]],
                rows_v.at[pl.ds(k * CHUNK, CHUNK)],
                sem,
            )
            for k in range(K)
        ]
        for cp in copies:
            cp.wait()
        pltpu.sync_copy(rows_v, out_hbm.at[pl.ds(off, K * CHUNK)])
        return carry

    lax.fori_loop(0, NITER, body, 0)


def kernel(input_ids, embedding, W, b):
    b2 = jnp.concatenate([b, b]).reshape(1, 128)
    packed = _transform(embedding, embedding, W, b2)
    ids = input_ids.astype(jnp.int32)
    flat = ids.reshape(ROWS)
    q = (flat // (2 * TBLK)) * TBLK + flat % TBLK
    gathered = _sc_gather(packed, q)
    right = (flat // TBLK) % 2
    g3 = gathered.reshape(B, L, 128)
    sel = (right == 1).reshape(B, L, 1)
    return jnp.where(sel, g3[:, :, EMBED:], g3[:, :, :EMBED])


# speedup vs baseline: 1.4463x; 1.4463x over previous
import functools

import jax
import jax.numpy as jnp
from jax import lax
from jax.experimental import pallas as pl
from jax.experimental.pallas import tpu as pltpu
from jax.experimental.pallas import tpu_sc as plsc

VOCAB = 1000000
EMBED = 64
B = 16384
L = 20
ROWS = B * L

TBLK = 2048
NBLOCKS = -(-VOCAB // (2 * TBLK))
PAIRS_PAD = NBLOCKS * TBLK
LANE_BLOCKS = -(-VOCAB // TBLK) - 1

NC = 2
NS = 16
NW = NC * NS
ROWS_PER_W = ROWS // NW
CHUNK = 128
K = 4
NITER = ROWS_PER_W // (K * CHUNK)

BBLK = 2048
NB = B // BBLK


def _tf_body(xlt_ref, xrt_ref, w_ref, b2_ref, o_ref):
    w = w_ref[...]
    yl = lax.dot_general(xlt_ref[...], w, (((0,), (1,)), ((), ())),
                         preferred_element_type=jnp.float32)
    yr = lax.dot_general(xrt_ref[...], w, (((0,), (1,)), ((), ())),
                         preferred_element_type=jnp.float32)
    o_ref[...] = jnp.concatenate([yl, yr], axis=1) + b2_ref[...]


_transform = pl.pallas_call(
    _tf_body,
    grid=(NBLOCKS,),
    in_specs=[
        pl.BlockSpec((EMBED, TBLK), lambda i: (0, 2 * i)),
        pl.BlockSpec((EMBED, TBLK), lambda i: (0, jnp.minimum(2 * i + 1,
                                                              LANE_BLOCKS))),
        pl.BlockSpec((EMBED, EMBED), lambda i: (0, 0)),
        pl.BlockSpec((1, 128), lambda i: (0, 0)),
    ],
    out_specs=pl.BlockSpec((TBLK, 128), lambda i: (i, 0)),
    out_shape=jax.ShapeDtypeStruct((PAIRS_PAD, 128), jnp.float32),
)

_mesh = plsc.VectorSubcoreMesh(core_axis_name="c", subcore_axis_name="s")


@functools.partial(
    pl.kernel,
    mesh=_mesh,
    out_type=jax.ShapeDtypeStruct((ROWS, 128), jnp.float32),
    scratch_types=[
        pltpu.VMEM((K * CHUNK,), jnp.int32),
        pltpu.VMEM((K * CHUNK, 128), jnp.float32),
        pltpu.SemaphoreType.DMA,
    ],
    compiler_params=pltpu.CompilerParams(use_tc_tiling_on_sc=False),
)
def _sc_gather(table_hbm, idx_hbm, out_hbm, idx_v, rows_v, sem):
    wid = lax.axis_index("s") * NC + lax.axis_index("c")
    base = wid * ROWS_PER_W

    def body(i, carry):
        off = pl.multiple_of(base + i * (K * CHUNK), K * CHUNK)
        pltpu.sync_copy(idx_hbm.at[pl.ds(off, K * CHUNK)], idx_v)
        copies = [
            pltpu.async_copy(
                table_hbm.at[idx_v.at[pl.ds(k * CHUNK, CHUNK)]],
                rows_v.at[pl.ds(k * CHUNK, CHUNK)],
                sem,
            )
            for k in range(K)
        ]
        for cp in copies:
            cp.wait()
        pltpu.sync_copy(rows_v, out_hbm.at[pl.ds(off, K * CHUNK)])
        return carry

    lax.fori_loop(0, NITER, body, 0)


def _sel_body(g_ref, p_ref, o_ref):
    g = g_ref[...]
    p = p_ref[...]
    sel = jnp.where(p == 1, g[:, EMBED:], g[:, :EMBED])
    o_ref[...] = sel.T[None]


_select_t = pl.pallas_call(
    _sel_body,
    grid=(L, NB),
    in_specs=[
        pl.BlockSpec((BBLK, 128), lambda l, j: (l * NB + j, 0)),
        pl.BlockSpec((BBLK, 1), lambda l, j: (l * NB + j, 0)),
    ],
    out_specs=pl.BlockSpec((1, EMBED, BBLK), lambda l, j: (l, 0, j)),
    out_shape=jax.ShapeDtypeStruct((L, EMBED, B), jnp.float32),
)


def kernel(input_ids, embedding, W, b):
    b2 = jnp.concatenate([b, b]).reshape(1, 128)
    packed = _transform(embedding.T, embedding.T, W, b2)
    ids = input_ids.astype(jnp.int32)
    flat = ids.T.reshape(ROWS)
    q = (flat // (2 * TBLK)) * TBLK + flat % TBLK
    gathered = _sc_gather(packed, q)
    par = ((flat // TBLK) % 2).reshape(ROWS, 1)
    out_t = _select_t(gathered, par)
    return jnp.transpose(out_t, (2, 0, 1))


# speedup vs baseline: 1.6508x; 1.1414x over previous
import functools

import jax
import jax.numpy as jnp
from jax import lax
from jax.experimental import pallas as pl
from jax.experimental.pallas import tpu as pltpu
from jax.experimental.pallas import tpu_sc as plsc

VOCAB = 1000000
EMBED = 64
B = 16384
L = 20
ROWS = B * L

TBLK = 4096
NBLOCKS = -(-VOCAB // (2 * TBLK))
PAIRS_PAD = NBLOCKS * TBLK
LANE_BLOCKS = -(-VOCAB // TBLK) - 1

NC = 2
NS = 16
NW = NC * NS
ROWS_PER_W = ROWS // NW
CHUNK = 128
K = 4
NITER = ROWS_PER_W // (K * CHUNK)

BBLK = 4096
NB = B // BBLK


def _tf_body(xlt_ref, xrt_ref, w_ref, b2_ref, o_ref):
    w = w_ref[...]
    yl = lax.dot_general(xlt_ref[...], w, (((0,), (1,)), ((), ())),
                         preferred_element_type=jnp.float32)
    yr = lax.dot_general(xrt_ref[...], w, (((0,), (1,)), ((), ())),
                         preferred_element_type=jnp.float32)
    o_ref[...] = jnp.concatenate([yl, yr], axis=1) + b2_ref[...]


_transform = pl.pallas_call(
    _tf_body,
    grid=(NBLOCKS,),
    in_specs=[
        pl.BlockSpec((EMBED, TBLK), lambda i: (0, 2 * i)),
        pl.BlockSpec((EMBED, TBLK), lambda i: (0, jnp.minimum(2 * i + 1,
                                                              LANE_BLOCKS))),
        pl.BlockSpec((EMBED, EMBED), lambda i: (0, 0)),
        pl.BlockSpec((1, 128), lambda i: (0, 0)),
    ],
    out_specs=pl.BlockSpec((TBLK, 128), lambda i: (i, 0)),
    out_shape=jax.ShapeDtypeStruct((PAIRS_PAD, 128), jnp.float32),
)

_mesh = plsc.VectorSubcoreMesh(core_axis_name="c", subcore_axis_name="s")


@functools.partial(
    pl.kernel,
    mesh=_mesh,
    out_type=jax.ShapeDtypeStruct((ROWS, 128), jnp.float32),
    scratch_types=[
        pltpu.VMEM((K * CHUNK,), jnp.int32),
        pltpu.VMEM((K * CHUNK, 128), jnp.float32),
        pltpu.SemaphoreType.DMA,
    ],
    compiler_params=pltpu.CompilerParams(use_tc_tiling_on_sc=False),
)
def _sc_gather(table_hbm, idx_hbm, out_hbm, idx_v, rows_v, sem):
    wid = lax.axis_index("s") * NC + lax.axis_index("c")
    base = wid * ROWS_PER_W

    def body(i, carry):
        off = pl.multiple_of(base + i * (K * CHUNK), K * CHUNK)
        pltpu.sync_copy(idx_hbm.at[pl.ds(off, K * CHUNK)], idx_v)
        copies = [
            pltpu.async_copy(
                table_hbm.at[idx_v.at[pl.ds(k * CHUNK, CHUNK)]],
                rows_v.at[pl.ds(k * CHUNK, CHUNK)],
                sem,
            )
            for k in range(K)
        ]
        for cp in copies:
            cp.wait()
        pltpu.sync_copy(rows_v, out_hbm.at[pl.ds(off, K * CHUNK)])
        return carry

    lax.fori_loop(0, NITER, body, 0)


def _sel_body(g_ref, p_ref, o_ref):
    g = g_ref[...]
    p = p_ref[...]
    sel = jnp.where(p == 1, g[:, EMBED:], g[:, :EMBED])
    o_ref[...] = sel.T[None]


_select_t = pl.pallas_call(
    _sel_body,
    grid=(L, NB),
    in_specs=[
        pl.BlockSpec((BBLK, 128), lambda l, j: (l * NB + j, 0)),
        pl.BlockSpec((BBLK, 1), lambda l, j: (l * NB + j, 0)),
    ],
    out_specs=pl.BlockSpec((1, EMBED, BBLK), lambda l, j: (l, 0, j)),
    out_shape=jax.ShapeDtypeStruct((L, EMBED, B), jnp.float32),
)


def kernel(input_ids, embedding, W, b):
    b2 = jnp.concatenate([b, b]).reshape(1, 128)
    packed = _transform(embedding.T, embedding.T, W, b2)
    ids = input_ids.astype(jnp.int32)
    flat = ids.T.reshape(ROWS)
    q = (flat // (2 * TBLK)) * TBLK + flat % TBLK
    gathered = _sc_gather(packed, q)
    par = ((flat // TBLK) % 2).reshape(ROWS, 1)
    out_t = _select_t(gathered, par)
    return jnp.transpose(out_t, (2, 0, 1))


# speedup vs baseline: 1.7748x; 1.0751x over previous
import functools

import jax
import jax.numpy as jnp
from jax import lax
from jax.experimental import pallas as pl
from jax.experimental.pallas import tpu as pltpu
from jax.experimental.pallas import tpu_sc as plsc

VOCAB = 1000000
EMBED = 64
B = 16384
L = 20
ROWS = B * L

TBLK = 8192
NBLOCKS = -(-VOCAB // (2 * TBLK))
PAIRS_PAD = NBLOCKS * TBLK
LANE_BLOCKS = -(-VOCAB // TBLK) - 1

NC = 2
NS = 16
NW = NC * NS
ROWS_PER_W = ROWS // NW
CHUNK = 128
K = 4
NITER = ROWS_PER_W // (K * CHUNK)

BBLK = 8192
NB = B // BBLK


def _tf_body(xlt_ref, xrt_ref, w_ref, b2_ref, o_ref):
    w = w_ref[...]
    yl = lax.dot_general(xlt_ref[...], w, (((0,), (1,)), ((), ())),
                         preferred_element_type=jnp.float32)
    yr = lax.dot_general(xrt_ref[...], w, (((0,), (1,)), ((), ())),
                         preferred_element_type=jnp.float32)
    o_ref[...] = jnp.concatenate([yl, yr], axis=1) + b2_ref[...]


_transform = pl.pallas_call(
    _tf_body,
    grid=(NBLOCKS,),
    in_specs=[
        pl.BlockSpec((EMBED, TBLK), lambda i: (0, 2 * i)),
        pl.BlockSpec((EMBED, TBLK), lambda i: (0, jnp.minimum(2 * i + 1,
                                                              LANE_BLOCKS))),
        pl.BlockSpec((EMBED, EMBED), lambda i: (0, 0)),
        pl.BlockSpec((1, 128), lambda i: (0, 0)),
    ],
    out_specs=pl.BlockSpec((TBLK, 128), lambda i: (i, 0)),
    out_shape=jax.ShapeDtypeStruct((PAIRS_PAD, 128), jnp.float32),
)

_mesh = plsc.VectorSubcoreMesh(core_axis_name="c", subcore_axis_name="s")


@functools.partial(
    pl.kernel,
    mesh=_mesh,
    out_type=jax.ShapeDtypeStruct((ROWS, 128), jnp.float32),
    scratch_types=[
        pltpu.VMEM((K * CHUNK,), jnp.int32),
        pltpu.VMEM((K * CHUNK, 128), jnp.float32),
        pltpu.SemaphoreType.DMA,
    ],
    compiler_params=pltpu.CompilerParams(use_tc_tiling_on_sc=False),
)
def _sc_gather(table_hbm, idx_hbm, out_hbm, idx_v, rows_v, sem):
    wid = lax.axis_index("s") * NC + lax.axis_index("c")
    base = wid * ROWS_PER_W

    def body(i, carry):
        off = pl.multiple_of(base + i * (K * CHUNK), K * CHUNK)
        pltpu.sync_copy(idx_hbm.at[pl.ds(off, K * CHUNK)], idx_v)
        copies = [
            pltpu.async_copy(
                table_hbm.at[idx_v.at[pl.ds(k * CHUNK, CHUNK)]],
                rows_v.at[pl.ds(k * CHUNK, CHUNK)],
                sem,
            )
            for k in range(K)
        ]
        for cp in copies:
            cp.wait()
        pltpu.sync_copy(rows_v, out_hbm.at[pl.ds(off, K * CHUNK)])
        return carry

    lax.fori_loop(0, NITER, body, 0)


def _sel_body(g_ref, p_ref, o_ref):
    g = g_ref[...]
    p = p_ref[...]
    sel = jnp.where(p == 1, g[:, EMBED:], g[:, :EMBED])
    o_ref[...] = sel.T[None]


_select_t = pl.pallas_call(
    _sel_body,
    grid=(L, NB),
    in_specs=[
        pl.BlockSpec((BBLK, 128), lambda l, j: (l * NB + j, 0)),
        pl.BlockSpec((BBLK, 1), lambda l, j: (l * NB + j, 0)),
    ],
    out_specs=pl.BlockSpec((1, EMBED, BBLK), lambda l, j: (l, 0, j)),
    out_shape=jax.ShapeDtypeStruct((L, EMBED, B), jnp.float32),
)


def kernel(input_ids, embedding, W, b):
    b2 = jnp.concatenate([b, b]).reshape(1, 128)
    packed = _transform(embedding.T, embedding.T, W, b2)
    ids = input_ids.astype(jnp.int32)
    flat = ids.T.reshape(ROWS)
    q = (flat // (2 * TBLK)) * TBLK + flat % TBLK
    gathered = _sc_gather(packed, q)
    par = ((flat // TBLK) % 2).reshape(ROWS, 1)
    out_t = _select_t(gathered, par)
    return jnp.transpose(out_t, (2, 0, 1))


# speedup vs baseline: 1.8092x; 1.0194x over previous
import functools

import jax
import jax.numpy as jnp
from jax import lax
from jax.experimental import pallas as pl
from jax.experimental.pallas import tpu as pltpu
from jax.experimental.pallas import tpu_sc as plsc

VOCAB = 1000000
EMBED = 64
B = 16384
L = 20
ROWS = B * L

TBLK = 16384
NBLOCKS = -(-VOCAB // (2 * TBLK))
PAIRS_PAD = NBLOCKS * TBLK
LANE_BLOCKS = -(-VOCAB // TBLK) - 1

NC = 2
NS = 16
NW = NC * NS
ROWS_PER_W = ROWS // NW
CHUNK = 128
K = 4
NITER = ROWS_PER_W // (K * CHUNK)

BBLK = 16384
NB = B // BBLK


def _tf_body(xlt_ref, xrt_ref, w_ref, b2_ref, o_ref):
    w = w_ref[...]
    yl = lax.dot_general(xlt_ref[...], w, (((0,), (1,)), ((), ())),
                         preferred_element_type=jnp.float32)
    yr = lax.dot_general(xrt_ref[...], w, (((0,), (1,)), ((), ())),
                         preferred_element_type=jnp.float32)
    o_ref[...] = jnp.concatenate([yl, yr], axis=1) + b2_ref[...]


_transform = pl.pallas_call(
    _tf_body,
    grid=(NBLOCKS,),
    in_specs=[
        pl.BlockSpec((EMBED, TBLK), lambda i: (0, 2 * i)),
        pl.BlockSpec((EMBED, TBLK), lambda i: (0, jnp.minimum(2 * i + 1,
                                                              LANE_BLOCKS))),
        pl.BlockSpec((EMBED, EMBED), lambda i: (0, 0)),
        pl.BlockSpec((1, 128), lambda i: (0, 0)),
    ],
    out_specs=pl.BlockSpec((TBLK, 128), lambda i: (i, 0)),
    out_shape=jax.ShapeDtypeStruct((PAIRS_PAD, 128), jnp.float32),
)

_mesh = plsc.VectorSubcoreMesh(core_axis_name="c", subcore_axis_name="s")


@functools.partial(
    pl.kernel,
    mesh=_mesh,
    out_type=jax.ShapeDtypeStruct((ROWS, 128), jnp.float32),
    scratch_types=[
        pltpu.VMEM((K * CHUNK,), jnp.int32),
        pltpu.VMEM((K * CHUNK, 128), jnp.float32),
        pltpu.SemaphoreType.DMA,
    ],
    compiler_params=pltpu.CompilerParams(use_tc_tiling_on_sc=False),
)
def _sc_gather(table_hbm, idx_hbm, out_hbm, idx_v, rows_v, sem):
    wid = lax.axis_index("s") * NC + lax.axis_index("c")
    base = wid * ROWS_PER_W

    def body(i, carry):
        off = pl.multiple_of(base + i * (K * CHUNK), K * CHUNK)
        pltpu.sync_copy(idx_hbm.at[pl.ds(off, K * CHUNK)], idx_v)
        copies = [
            pltpu.async_copy(
                table_hbm.at[idx_v.at[pl.ds(k * CHUNK, CHUNK)]],
                rows_v.at[pl.ds(k * CHUNK, CHUNK)],
                sem,
            )
            for k in range(K)
        ]
        for cp in copies:
            cp.wait()
        pltpu.sync_copy(rows_v, out_hbm.at[pl.ds(off, K * CHUNK)])
        return carry

    lax.fori_loop(0, NITER, body, 0)


def _sel_body(g_ref, p_ref, o_ref):
    g = g_ref[...]
    p = p_ref[...]
    sel = jnp.where(p == 1, g[:, EMBED:], g[:, :EMBED])
    o_ref[...] = sel.T[None]


_select_t = pl.pallas_call(
    _sel_body,
    grid=(L, NB),
    in_specs=[
        pl.BlockSpec((BBLK, 128), lambda l, j: (l * NB + j, 0)),
        pl.BlockSpec((BBLK, 1), lambda l, j: (l * NB + j, 0)),
    ],
    out_specs=pl.BlockSpec((1, EMBED, BBLK), lambda l, j: (l, 0, j)),
    out_shape=jax.ShapeDtypeStruct((L, EMBED, B), jnp.float32),
)


def kernel(input_ids, embedding, W, b):
    b2 = jnp.concatenate([b, b]).reshape(1, 128)
    packed = _transform(embedding.T, embedding.T, W, b2)
    ids = input_ids.astype(jnp.int32)
    flat = ids.T.reshape(ROWS)
    q = (flat // (2 * TBLK)) * TBLK + flat % TBLK
    gathered = _sc_gather(packed, q)
    par = ((flat // TBLK) % 2).reshape(ROWS, 1)
    out_t = _select_t(gathered, par)
    return jnp.transpose(out_t, (2, 0, 1))


# speedup vs baseline: 2.7269x; 1.5073x over previous
import functools

import jax
import jax.numpy as jnp
from jax import lax
from jax.experimental import pallas as pl
from jax.experimental.pallas import tpu as pltpu
from jax.experimental.pallas import tpu_sc as plsc

VOCAB = 1000000
EMBED = 64
B = 16384
L = 20
ROWS = B * L

TBLK = 16384
NBLOCKS = -(-VOCAB // (2 * TBLK))
PAIRS_PAD = NBLOCKS * TBLK
LANE_BLOCKS = -(-VOCAB // TBLK) - 1

NC = 2
NS = 16
NW = NC * NS
ROWS_PER_W = ROWS // NW
CHUNK = 128
K = 4
NITER = ROWS_PER_W // (K * CHUNK)

BBLK = 16384
NB = B // BBLK


def _tf_body(xlt_ref, xrt_ref, w_ref, b2_ref, o_ref):
    w = w_ref[...]
    yl = lax.dot_general(xlt_ref[...], w, (((0,), (1,)), ((), ())),
                         preferred_element_type=jnp.float32)
    yr = lax.dot_general(xrt_ref[...], w, (((0,), (1,)), ((), ())),
                         preferred_element_type=jnp.float32)
    o_ref[...] = jnp.concatenate([yl, yr], axis=1) + b2_ref[...]


_transform = pl.pallas_call(
    _tf_body,
    grid=(NBLOCKS,),
    in_specs=[
        pl.BlockSpec((EMBED, TBLK), lambda i: (0, 2 * i)),
        pl.BlockSpec((EMBED, TBLK), lambda i: (0, jnp.minimum(2 * i + 1,
                                                              LANE_BLOCKS))),
        pl.BlockSpec((EMBED, EMBED), lambda i: (0, 0)),
        pl.BlockSpec((1, 128), lambda i: (0, 0)),
    ],
    out_specs=pl.BlockSpec((TBLK, 128), lambda i: (i, 0)),
    out_shape=jax.ShapeDtypeStruct((PAIRS_PAD, 128), jnp.float32),
)

_mesh = plsc.VectorSubcoreMesh(core_axis_name="c", subcore_axis_name="s")


@functools.partial(
    pl.kernel,
    mesh=_mesh,
    out_type=jax.ShapeDtypeStruct((ROWS, EMBED), jnp.float32),
    scratch_types=[
        pltpu.VMEM((K * CHUNK,), jnp.int32),
        pltpu.VMEM((K * CHUNK, EMBED), jnp.float32),
        pltpu.SemaphoreType.DMA,
    ],
    compiler_params=pltpu.CompilerParams(use_tc_tiling_on_sc=False),
)
def _sc_gather(table_hbm, idx_hbm, out_hbm, idx_v, rows_v, sem):
    wid = lax.axis_index("s") * NC + lax.axis_index("c")
    base = wid * ROWS_PER_W

    def body(i, carry):
        off = pl.multiple_of(base + i * (K * CHUNK), K * CHUNK)
        pltpu.sync_copy(idx_hbm.at[pl.ds(off, K * CHUNK)], idx_v)
        copies = [
            pltpu.async_copy(
                table_hbm.at[idx_v.at[pl.ds(k * CHUNK, CHUNK)]],
                rows_v.at[pl.ds(k * CHUNK, CHUNK)],
                sem,
            )
            for k in range(K)
        ]
        for cp in copies:
            cp.wait()
        pltpu.sync_copy(rows_v, out_hbm.at[pl.ds(off, K * CHUNK)])
        return carry

    lax.fori_loop(0, NITER, body, 0)


def _tr_body(g_ref, o_ref):
    g = g_ref[...]
    o_ref[...] = jnp.concatenate([g[:, :EMBED].T, g[:, EMBED:].T],
                                 axis=1)[None]


_tail_t = pl.pallas_call(
    _tr_body,
    grid=(L,),
    in_specs=[pl.BlockSpec((B // 2, 128), lambda l: (l, 0))],
    out_specs=pl.BlockSpec((1, EMBED, B), lambda l: (l, 0, 0)),
    out_shape=jax.ShapeDtypeStruct((L, EMBED, B), jnp.float32),
)


def kernel(input_ids, embedding, W, b):
    b2 = jnp.concatenate([b, b]).reshape(1, 128)
    packed = _transform(embedding.T, embedding.T, W, b2)
    half = packed.reshape(2 * PAIRS_PAD, EMBED)
    ids = input_ids.astype(jnp.int32).T
    flat = jnp.stack([ids[:, :B // 2], ids[:, B // 2:]], axis=-1).reshape(ROWS)
    r = 2 * ((flat // (2 * TBLK)) * TBLK + flat % TBLK) + (flat // TBLK) % 2
    gathered = _sc_gather(half, r)
    out_t = _tail_t(gathered.reshape(ROWS // 2, 128))
    return jnp.transpose(out_t, (2, 0, 1))


# speedup vs baseline: 2.7367x; 1.0036x over previous
import functools

import jax
import jax.numpy as jnp
from jax import lax
from jax.experimental import pallas as pl
from jax.experimental.pallas import tpu as pltpu
from jax.experimental.pallas import tpu_sc as plsc

VOCAB = 1000000
EMBED = 64
B = 16384
L = 20
ROWS = B * L

TBLK = 16384
NBLOCKS = -(-VOCAB // (2 * TBLK))
PAIRS_PAD = NBLOCKS * TBLK
LANE_BLOCKS = -(-VOCAB // TBLK) - 1

NC = 2
NS = 16
NW = NC * NS
ROWS_PER_W = ROWS // NW
CHUNK = 128
K = 4
NITER = ROWS_PER_W // (K * CHUNK)


def _tf_body(xlt_ref, xrt_ref, w_ref, b2_ref, o_ref):
    w = w_ref[...]
    yl = lax.dot_general(xlt_ref[...], w, (((0,), (1,)), ((), ())),
                         preferred_element_type=jnp.float32)
    yr = lax.dot_general(xrt_ref[...], w, (((0,), (1,)), ((), ())),
                         preferred_element_type=jnp.float32)
    o_ref[...] = jnp.concatenate([yl, yr], axis=1) + b2_ref[...]


_transform = pl.pallas_call(
    _tf_body,
    grid=(NBLOCKS,),
    in_specs=[
        pl.BlockSpec((EMBED, TBLK), lambda i: (0, 2 * i)),
        pl.BlockSpec((EMBED, TBLK), lambda i: (0, jnp.minimum(2 * i + 1,
                                                              LANE_BLOCKS))),
        pl.BlockSpec((EMBED, EMBED), lambda i: (0, 0)),
        pl.BlockSpec((1, 128), lambda i: (0, 0)),
    ],
    out_specs=pl.BlockSpec((TBLK, 128), lambda i: (i, 0)),
    out_shape=jax.ShapeDtypeStruct((PAIRS_PAD, 128), jnp.float32),
)

_mesh = plsc.VectorSubcoreMesh(core_axis_name="c", subcore_axis_name="s")


@functools.partial(
    pl.kernel,
    mesh=_mesh,
    out_type=jax.ShapeDtypeStruct((ROWS, EMBED), jnp.float32),
    scratch_types=[
        pltpu.VMEM((K * CHUNK,), jnp.int32),
        pltpu.VMEM((K * CHUNK, EMBED), jnp.float32),
        pltpu.SemaphoreType.DMA,
    ],
    compiler_params=pltpu.CompilerParams(use_tc_tiling_on_sc=False),
)
def _sc_gather(table_hbm, idx_hbm, out_hbm, idx_v, rows_v, sem):
    wid = lax.axis_index("s") * NC + lax.axis_index("c")
    base = wid * ROWS_PER_W

    def body(i, carry):
        off = pl.multiple_of(base + i * (K * CHUNK), K * CHUNK)
        pltpu.sync_copy(idx_hbm.at[pl.ds(off, K * CHUNK)], idx_v)
        copies = [
            pltpu.async_copy(
                table_hbm.at[idx_v.at[pl.ds(k * CHUNK, CHUNK)]],
                rows_v.at[pl.ds(k * CHUNK, CHUNK)],
                sem,
            )
            for k in range(K)
        ]
        for cp in copies:
            cp.wait()
        pltpu.sync_copy(rows_v, out_hbm.at[pl.ds(off, K * CHUNK)])
        return carry

    lax.fori_loop(0, NITER, body, 0)


def _tr_body(g_ref, o_ref):
    g = g_ref[...]
    o_ref[...] = jnp.concatenate([g[:, :EMBED].T, g[:, EMBED:].T],
                                 axis=1)[None]


_tail_t = pl.pallas_call(
    _tr_body,
    grid=(L,),
    in_specs=[pl.BlockSpec((B // 2, 128), lambda l: (l, 0))],
    out_specs=pl.BlockSpec((1, EMBED, B), lambda l: (l, 0, 0)),
    out_shape=jax.ShapeDtypeStruct((L, EMBED, B), jnp.float32),
)


def kernel(input_ids, embedding, W, b):
    b2 = jnp.concatenate([b, b]).reshape(1, 128)
    packed = _transform(embedding.T, embedding.T, W, b2)
    half = packed.reshape(2 * PAIRS_PAD, EMBED)
    ids = input_ids.astype(jnp.int32).T
    flat = jnp.stack([ids[:, :B // 2], ids[:, B // 2:]], axis=-1).reshape(ROWS)
    r = 2 * ((flat // (2 * TBLK)) * TBLK + flat % TBLK) + (flat // TBLK) % 2
    gathered = _sc_gather(half, r)
    out_t = _tail_t(gathered.reshape(ROWS // 2, 128))
    return jnp.transpose(out_t, (2, 0, 1))
